# baseline (speedup 1.0000x reference)
import jax
import jax.numpy as jnp
from jax.experimental import pallas as pl
from jax.experimental.pallas import tpu as pltpu

_N_CHUNKS = 16


def _identity_copy_kernel(x_ref, o_ref, sems):
    rows = x_ref.shape[0]
    chunk = rows // _N_CHUNKS
    for i in range(_N_CHUNKS):
        pltpu.make_async_copy(
            x_ref.at[pl.ds(i * chunk, chunk)],
            o_ref.at[pl.ds(i * chunk, chunk)],
            sems.at[i],
        ).start()
    for i in range(_N_CHUNKS):
        pltpu.make_async_copy(
            x_ref.at[pl.ds(i * chunk, chunk)],
            o_ref.at[pl.ds(i * chunk, chunk)],
            sems.at[i],
        ).wait()


def kernel(x, rel_pos_bias):
    del rel_pos_bias
    b, s, d = x.shape
    x2 = x.reshape(b * s, d)
    out = pl.pallas_call(
        _identity_copy_kernel,
        out_shape=jax.ShapeDtypeStruct((b * s, d), x.dtype),
        in_specs=[pl.BlockSpec(memory_space=pl.ANY)],
        out_specs=pl.BlockSpec(memory_space=pl.ANY),
        scratch_shapes=[pltpu.SemaphoreType.DMA((_N_CHUNKS,))],
    )(x2)
    return out.reshape(b, s, d)


# speedup vs baseline: 48.1740x; 48.1740x over previous
import jax
import jax.numpy as jnp
from jax.experimental import pallas as pl
from jax.experimental.pallas import tpu as pltpu

_BLOCK_ROWS = 512


def _identity_copy_kernel(x_ref, o_ref):
    o_ref[...] = x_ref[...]


def kernel(x, rel_pos_bias):
    del rel_pos_bias
    b, s, d = x.shape
    rows = b * s
    x2 = x.reshape(rows, d)
    grid = rows // _BLOCK_ROWS
    out = pl.pallas_call(
        _identity_copy_kernel,
        out_shape=jax.ShapeDtypeStruct((rows, d), x.dtype),
        grid=(grid,),
        in_specs=[pl.BlockSpec((_BLOCK_ROWS, d), lambda i: (i, 0))],
        out_specs=pl.BlockSpec((_BLOCK_ROWS, d), lambda i: (i, 0)),
    )(x2)
    return out.reshape(b, s, d)


# speedup vs baseline: 49.0845x; 1.0189x over previous
import jax
import jax.numpy as jnp
from jax.experimental import pallas as pl
from jax.experimental.pallas import tpu as pltpu

_BLOCK_ROWS = 1024


def _identity_copy_kernel(x_ref, o_ref):
    o_ref[...] = x_ref[...]


def kernel(x, rel_pos_bias):
    del rel_pos_bias
    b, s, d = x.shape
    rows = b * s
    x2 = x.reshape(rows, d)
    grid = rows // _BLOCK_ROWS
    out = pl.pallas_call(
        _identity_copy_kernel,
        out_shape=jax.ShapeDtypeStruct((rows, d), x.dtype),
        grid=(grid,),
        in_specs=[pl.BlockSpec((_BLOCK_ROWS, d), lambda i: (i, 0))],
        out_specs=pl.BlockSpec((_BLOCK_ROWS, d), lambda i: (i, 0)),
    )(x2)
    return out.reshape(b, s, d)
